# baseline (speedup 1.0000x reference)
import functools

import jax
import jax.numpy as jnp
from jax import lax
from jax.experimental import pallas as pl
from jax.experimental.pallas import tpu as pltpu
from jax.experimental.pallas import tpu_sc as plsc

N_NODES = 10000
D_FEAT = 256
HIDDEN = 256
N_CLASSES = 64
N_EDGES = 160000

NC, NS, L = 2, 16, 16
CHUNK = 128
E_PAD = 163840
ACC_ROWS = 10240
JUNK_ROW = 10008
ROWS_PER_TILE = ACC_ROWS // NS
ACC1_ROWS = 10016
ROWS1_PER_TILE = ACC1_ROWS // NS
CW = 64
K1 = 8
K2 = 8

N_CHUNKS_1 = E_PAD // NS // CHUNK
N_CHUNKS_2 = E_PAD // (NC * NS) // CHUNK


def _fill(ref, n_rows, n_cols, val):
    groups = n_cols // L

    def body(i, _):
        r = i // groups
        g = i % groups
        ref[r, pl.ds(g * L, L)] = jnp.full((L,), val, jnp.float32)
        return 0

    lax.fori_loop(0, n_rows * groups, body, 0)


def _mesh():
    return plsc.VectorSubcoreMesh(core_axis_name="c", subcore_axis_name="s")


_SC_PARAMS = pltpu.CompilerParams(use_tc_tiling_on_sc=False)


@functools.partial(
    pl.kernel,
    out_type=[
        jax.ShapeDtypeStruct((ACC1_ROWS, CW), jnp.float32),
        jax.ShapeDtypeStruct((ACC1_ROWS, CW), jnp.float32),
        jax.ShapeDtypeStruct((ACC1_ROWS, CW), jnp.float32),
        jax.ShapeDtypeStruct((ACC1_ROWS, CW), jnp.float32),
        jax.ShapeDtypeStruct((ACC1_ROWS, L), jnp.float32),
        jax.ShapeDtypeStruct((ACC1_ROWS, L), jnp.float32),
    ],
    mesh=_mesh(),
    compiler_params=_SC_PARAMS,
    scratch_types=[
        pltpu.VMEM((2, K1, CHUNK), jnp.int32),
        pltpu.VMEM((2, K1, CHUNK), jnp.int32),
        pltpu.VMEM((K1, CHUNK, CW), jnp.float32),
        pltpu.VMEM((CHUNK, L), jnp.float32),
        pltpu.VMEM_SHARED((ACC1_ROWS, CW), jnp.float32),
        pltpu.VMEM_SHARED((ACC1_ROWS, L), jnp.float32),
        pltpu.SemaphoreType.DMA((2,)),
        pltpu.SemaphoreType.DMA((2,)),
        pltpu.SemaphoreType.DMA((K1,)),
        pltpu.SemaphoreType.DMA((K1,)),
        pltpu.SemaphoreType.DMA,
    ],
)
def _sc_agg1(x0, x1, x2, x3, srcp, dstp, o0, o1, o2, o3, outc0, outc1,
             sidx, didx, rows, ones, acc, cacc, isems, idems, gsem, ssem, csem):
    c = lax.axis_index("c")
    s = lax.axis_index("s")

    row0 = s * ROWS1_PER_TILE
    sl = pl.ds(row0, ROWS1_PER_TILE)
    tables = (x0, x1, x2, x3)
    outs = (o0, o1, o2, o3)
    n_chunks = E_PAD // NS // CHUNK
    n_groups = n_chunks // K1
    half = n_groups // 2
    crow0 = s * n_chunks
    tail = ROWS1_PER_TILE - 4 * CHUNK

    for j in range(2):
        _fill(rows.at[0], CHUNK, CW, 0.0)
        if j == 0:
            _fill(ones, CHUNK, L, 0.0)
        for k in range(4):
            pltpu.sync_copy(rows.at[0], acc.at[pl.ds(row0 + k * CHUNK, CHUNK)])
            if j == 0:
                pltpu.sync_copy(ones, cacc.at[pl.ds(row0 + k * CHUNK, CHUNK)])
        pltpu.sync_copy(rows.at[0, pl.ds(0, tail)],
                        acc.at[pl.ds(row0 + 4 * CHUNK, tail)])
        if j == 0:
            pltpu.sync_copy(ones.at[pl.ds(0, tail)],
                            cacc.at[pl.ds(row0 + 4 * CHUNK, tail)])
            _fill(ones, CHUNK, L, 1.0)

        pltpu.async_copy(srcp.at[pl.ds(crow0, K1)], sidx.at[0], isems.at[0])
        pltpu.async_copy(dstp.at[pl.ds(crow0, K1)], didx.at[0], idems.at[0])
        plsc.subcore_barrier()

        def group(g, _):
            parity = lax.rem(g, 2)
            nxt = 1 - parity
            cnt_here = ((c == 0) & (g < half)) | ((c == 1) & (g >= half))

            @pl.when(g > 0)
            def _():
                for b in range(K1):
                    pltpu.make_async_copy(
                        rows.at[b], acc.at[didx.at[0, 0]], ssem.at[b]).wait()

            @pl.when(g + 1 < n_groups)
            def _():
                r = crow0 + (g + 1) * K1
                pltpu.async_copy(srcp.at[pl.ds(r, K1)], sidx.at[nxt],
                                 isems.at[nxt])
                pltpu.async_copy(dstp.at[pl.ds(r, K1)], didx.at[nxt],
                                 idems.at[nxt])

            pltpu.make_async_copy(
                srcp.at[pl.ds(0, K1)], sidx.at[parity], isems.at[parity]).wait()
            pltpu.make_async_copy(
                dstp.at[pl.ds(0, K1)], didx.at[parity], idems.at[parity]).wait()

            for b in range(K1):
                @pl.when(c == 0)
                def _():
                    pltpu.async_copy(
                        tables[j].at[sidx.at[parity, b]], rows.at[b],
                        gsem.at[b])

                @pl.when(c == 1)
                def _():
                    pltpu.async_copy(
                        tables[2 + j].at[sidx.at[parity, b]], rows.at[b],
                        gsem.at[b])

            for b in range(K1):
                pltpu.make_async_copy(
                    tables[j].at[sidx.at[parity, b]], rows.at[b],
                    gsem.at[b]).wait()
                pltpu.async_copy(
                    rows.at[b], acc.at[didx.at[parity, b]], ssem.at[b],
                    add=True)

                if j == 0:
                    @pl.when(cnt_here)
                    def _():
                        pltpu.async_copy(
                            ones, cacc.at[didx.at[parity, b]], csem, add=True)

            if j == 0:
                @pl.when(cnt_here)
                def _():
                    for b in range(K1):
                        pltpu.make_async_copy(
                            ones, cacc.at[didx.at[parity, b]], csem).wait()

            return 0

        lax.fori_loop(0, n_groups, group, 0)
        for b in range(K1):
            pltpu.make_async_copy(
                rows.at[b], acc.at[didx.at[0, 0]], ssem.at[b]).wait()
        plsc.subcore_barrier()

        @pl.when(c == 0)
        def _():
            pltpu.sync_copy(acc.at[sl], outs[j].at[sl])

        @pl.when(c == 1)
        def _():
            pltpu.sync_copy(acc.at[sl], outs[2 + j].at[sl])

        if j == 0:
            @pl.when(c == 0)
            def _():
                pltpu.sync_copy(cacc.at[sl], outc0.at[sl])

            @pl.when(c == 1)
            def _():
                pltpu.sync_copy(cacc.at[sl], outc1.at[sl])


@functools.partial(
    pl.kernel,
    out_type=[
        jax.ShapeDtypeStruct((ACC_ROWS, N_CLASSES), jnp.float32),
        jax.ShapeDtypeStruct((ACC_ROWS, N_CLASSES), jnp.float32),
    ],
    mesh=_mesh(),
    compiler_params=_SC_PARAMS,
    scratch_types=[
        pltpu.VMEM((N_CHUNKS_2, CHUNK), jnp.int32),
        pltpu.VMEM((N_CHUNKS_2, CHUNK), jnp.int32),
        pltpu.VMEM((K2, CHUNK, N_CLASSES), jnp.float32),
        pltpu.VMEM((CHUNK, N_CLASSES), jnp.float32),
        pltpu.VMEM_SHARED((ACC_ROWS, N_CLASSES), jnp.float32),
        pltpu.SemaphoreType.DMA((K2,)),
        pltpu.SemaphoreType.DMA((K2,)),
    ],
)
def _sc_agg2(p, srcp, dstp, out0, out1, sidx, didx, rows, zb, acc, gsem, ssem):
    c = lax.axis_index("c")
    s = lax.axis_index("s")

    _fill(zb, CHUNK, N_CLASSES, 0.0)

    crow = pl.ds((c * NS + s) * N_CHUNKS_2, N_CHUNKS_2)
    pltpu.sync_copy(srcp.at[crow], sidx)
    pltpu.sync_copy(dstp.at[crow], didx)

    row0 = s * ROWS_PER_TILE
    for k in range(ROWS_PER_TILE // CHUNK):
        pltpu.sync_copy(zb, acc.at[pl.ds(row0 + k * CHUNK, CHUNK)])
    plsc.subcore_barrier()

    def group(g, _):
        for b in range(K2):
            i = g * K2 + b

            @pl.when(g > 0)
            def _():
                pltpu.make_async_copy(
                    rows.at[b], acc.at[didx.at[0]], ssem.at[b]).wait()

            pltpu.async_copy(p.at[sidx.at[i]], rows.at[b], gsem.at[b])

        for b in range(K2):
            i = g * K2 + b
            pltpu.make_async_copy(
                p.at[sidx.at[i]], rows.at[b], gsem.at[b]).wait()
            pltpu.async_copy(
                rows.at[b], acc.at[didx.at[i]], ssem.at[b], add=True)
        return 0

    lax.fori_loop(0, N_CHUNKS_2 // K2, group, 0)
    for b in range(K2):
        pltpu.make_async_copy(
            rows.at[b], acc.at[didx.at[0]], ssem.at[b]).wait()
    plsc.subcore_barrier()

    sl = pl.ds(row0, ROWS_PER_TILE)

    @pl.when(c == 0)
    def _():
        pltpu.sync_copy(acc.at[sl], out0.at[sl])

    @pl.when(c == 1)
    def _():
        pltpu.sync_copy(acc.at[sl], out1.at[sl])


BLK = 1000


def _tc_mid_body(o0, o1, o2, o3, c0, c1, x, w1l, w1r, b1, w2l, w2r, b2,
                 p_out, r2_out):
    c = jnp.maximum(c0[:, 0:1] + c1[:, 0:1], 1.0)
    mean = jnp.concatenate([o0[...], o1[...], o2[...], o3[...]], axis=1) / c
    h = mean @ w1l[...] + x[...] @ w1r[...] + b1[...]
    h = jnp.maximum(h, 0.0)
    p_out[...] = h @ w2l[...]
    r2_out[...] = h @ w2r[...] + b2[...]


def _tc_mid(o0, o1, o2, o3, cnt0, cnt1, x, w1l, w1r, b1, w2l, w2r, b2):
    full = lambda shape: pl.BlockSpec(shape, lambda i: (0, 0))
    rows = lambda shape: pl.BlockSpec(shape, lambda i: (i, 0))
    return pl.pallas_call(
        _tc_mid_body,
        grid=(N_NODES // BLK,),
        in_specs=[
            rows((BLK, CW)), rows((BLK, CW)), rows((BLK, CW)), rows((BLK, CW)),
            rows((BLK, L)), rows((BLK, L)), rows((BLK, D_FEAT)),
            full((D_FEAT, HIDDEN)), full((D_FEAT, HIDDEN)), full((1, HIDDEN)),
            full((HIDDEN, N_CLASSES)), full((HIDDEN, N_CLASSES)), full((1, N_CLASSES)),
        ],
        out_specs=[rows((BLK, N_CLASSES)), rows((BLK, N_CLASSES))],
        out_shape=[
            jax.ShapeDtypeStruct((N_NODES, N_CLASSES), jnp.float32),
            jax.ShapeDtypeStruct((N_NODES, N_CLASSES), jnp.float32),
        ],
    )(o0, o1, o2, o3, cnt0, cnt1, x, w1l, w1r, b1, w2l, w2r, b2)


def _tc_final_body(a0, a1, c0, c1, r2, out):
    c = jnp.maximum(c0[:, 0:1] + c1[:, 0:1], 1.0)
    z = (a0[...] + a1[...]) / c + r2[...]
    m = jnp.max(z, axis=1, keepdims=True)
    zs = z - m
    out[...] = zs - jnp.log(jnp.sum(jnp.exp(zs), axis=1, keepdims=True))


def _tc_final(a0, a1, cnt0, cnt1, r2):
    rows = lambda shape: pl.BlockSpec(shape, lambda i: (i, 0))
    return pl.pallas_call(
        _tc_final_body,
        grid=(N_NODES // BLK,),
        in_specs=[rows((BLK, N_CLASSES)), rows((BLK, N_CLASSES)),
                  rows((BLK, L)), rows((BLK, L)), rows((BLK, N_CLASSES))],
        out_specs=rows((BLK, N_CLASSES)),
        out_shape=jax.ShapeDtypeStruct((N_NODES, N_CLASSES), jnp.float32),
    )(a0, a1, cnt0, cnt1, r2)


def kernel(x, edge_index, W1l, W1r, b1, W2l, W2r, b2):
    src = edge_index[0].astype(jnp.int32)
    dst = edge_index[1].astype(jnp.int32)
    n_pad = E_PAD - N_EDGES
    srcp = jnp.concatenate([src, jnp.zeros((n_pad,), jnp.int32)])
    dstp = jnp.concatenate([dst, jnp.full((n_pad,), JUNK_ROW, jnp.int32)])
    srcp = srcp.reshape(E_PAD // CHUNK, CHUNK)
    dstp = dstp.reshape(E_PAD // CHUNK, CHUNK)

    xs = [x[:, i * CW:(i + 1) * CW] for i in range(4)]
    o0, o1, o2, o3, cnt0, cnt1 = _sc_agg1(*xs, srcp, dstp)

    p, r2 = _tc_mid(o0, o1, o2, o3, cnt0, cnt1, x, W1l, W1r,
                    b1.reshape(1, -1), W2l, W2r, b2.reshape(1, -1))

    a0, a1 = _sc_agg2(p, srcp, dstp)
    return _tc_final(a0, a1, cnt0, cnt1, r2)


# speedup vs baseline: 1.7818x; 1.7818x over previous
import functools

import jax
import jax.numpy as jnp
from jax import lax
from jax.experimental import pallas as pl
from jax.experimental.pallas import tpu as pltpu
from jax.experimental.pallas import tpu_sc as plsc

N_NODES = 10000
D_FEAT = 256
HIDDEN = 256
N_CLASSES = 64
N_EDGES = 160000

NC, NS, L = 2, 16, 16
CHUNK = 128
E_PAD = 163840
ACC_ROWS = 10240
JUNK_ROW = 10008
ROWS_PER_TILE = ACC_ROWS // NS
ACC1_ROWS = 10016
ROWS1_PER_TILE = ACC1_ROWS // NS
CW = 64
K1 = 8
K2 = 8

N_CHUNKS_1 = E_PAD // NS // CHUNK
N_CHUNKS_2 = E_PAD // (NC * NS) // CHUNK


def _fill(ref, n_rows, n_cols, val):
    lanes = 2 * L if ref.dtype == jnp.bfloat16 else L
    groups = n_cols // lanes

    def body(i, _):
        r = i // groups
        g = i % groups
        ref[r, pl.ds(g * lanes, lanes)] = jnp.full((lanes,), val, ref.dtype)
        return 0

    lax.fori_loop(0, n_rows * groups, body, 0)


def _mesh():
    return plsc.VectorSubcoreMesh(core_axis_name="c", subcore_axis_name="s")


_SC_PARAMS = pltpu.CompilerParams(use_tc_tiling_on_sc=False)


@functools.partial(
    pl.kernel,
    out_type=[
        jax.ShapeDtypeStruct((ACC1_ROWS, 128), jnp.bfloat16),
        jax.ShapeDtypeStruct((ACC1_ROWS, 128), jnp.bfloat16),
        jax.ShapeDtypeStruct((ACC1_ROWS, L), jnp.float32),
        jax.ShapeDtypeStruct((ACC1_ROWS, L), jnp.float32),
    ],
    mesh=_mesh(),
    compiler_params=_SC_PARAMS,
    scratch_types=[
        pltpu.VMEM((2, K1, CHUNK), jnp.int32),
        pltpu.VMEM((2, K1, CHUNK), jnp.int32),
        pltpu.VMEM((K1, CHUNK, 128), jnp.bfloat16),
        pltpu.VMEM((CHUNK, L), jnp.float32),
        pltpu.VMEM_SHARED((ACC1_ROWS, 128), jnp.bfloat16),
        pltpu.VMEM_SHARED((ACC1_ROWS, L), jnp.float32),
        pltpu.SemaphoreType.DMA((2,)),
        pltpu.SemaphoreType.DMA((2,)),
        pltpu.SemaphoreType.DMA((K1,)),
        pltpu.SemaphoreType.DMA((K1,)),
        pltpu.SemaphoreType.DMA,
    ],
)
def _sc_agg1(xa, xb, srcp, dstp, oa, ob, outc0, outc1,
             sidx, didx, rows, ones, acc, cacc, isems, idems, gsem, ssem, csem):
    c = lax.axis_index("c")
    s = lax.axis_index("s")

    row0 = s * ROWS1_PER_TILE
    sl = pl.ds(row0, ROWS1_PER_TILE)
    n_chunks = E_PAD // NS // CHUNK
    n_groups = n_chunks // K1
    half = n_groups // 2
    crow0 = s * n_chunks

    _fill(ones, CHUNK, L, 0.0)
    _fill(rows.at[0], CHUNK, 128, 0.0)
    for k in range(4):
        pltpu.sync_copy(rows.at[0], acc.at[pl.ds(row0 + k * CHUNK, CHUNK)])
        pltpu.sync_copy(ones, cacc.at[pl.ds(row0 + k * CHUNK, CHUNK)])
    pltpu.sync_copy(rows.at[0, pl.ds(0, ROWS1_PER_TILE - 4 * CHUNK)],
                    acc.at[pl.ds(row0 + 4 * CHUNK, ROWS1_PER_TILE - 4 * CHUNK)])
    pltpu.sync_copy(ones.at[pl.ds(0, ROWS1_PER_TILE - 4 * CHUNK)],
                    cacc.at[pl.ds(row0 + 4 * CHUNK, ROWS1_PER_TILE - 4 * CHUNK)])
    _fill(ones, CHUNK, L, 1.0)

    pltpu.async_copy(srcp.at[pl.ds(crow0, K1)], sidx.at[0], isems.at[0])
    pltpu.async_copy(dstp.at[pl.ds(crow0, K1)], didx.at[0], idems.at[0])
    plsc.subcore_barrier()

    def group(g, _):
        parity = lax.rem(g, 2)
        nxt = 1 - parity
        cnt_here = ((c == 0) & (g < half)) | ((c == 1) & (g >= half))

        @pl.when(g > 0)
        def _():
            for b in range(K1):
                pltpu.make_async_copy(
                    rows.at[b], acc.at[didx.at[0, 0]], ssem.at[b]).wait()

        @pl.when(g + 1 < n_groups)
        def _():
            r = crow0 + (g + 1) * K1
            pltpu.async_copy(srcp.at[pl.ds(r, K1)], sidx.at[nxt], isems.at[nxt])
            pltpu.async_copy(dstp.at[pl.ds(r, K1)], didx.at[nxt], idems.at[nxt])

        pltpu.make_async_copy(
            srcp.at[pl.ds(0, K1)], sidx.at[parity], isems.at[parity]).wait()
        pltpu.make_async_copy(
            dstp.at[pl.ds(0, K1)], didx.at[parity], idems.at[parity]).wait()

        for b in range(K1):
            @pl.when(c == 0)
            def _():
                pltpu.async_copy(
                    xa.at[sidx.at[parity, b]], rows.at[b], gsem.at[b])

            @pl.when(c == 1)
            def _():
                pltpu.async_copy(
                    xb.at[sidx.at[parity, b]], rows.at[b], gsem.at[b])

        for b in range(K1):
            pltpu.make_async_copy(
                xa.at[sidx.at[parity, b]], rows.at[b], gsem.at[b]).wait()
            pltpu.async_copy(
                rows.at[b], acc.at[didx.at[parity, b]], ssem.at[b], add=True)

            @pl.when(cnt_here)
            def _():
                pltpu.async_copy(
                    ones, cacc.at[didx.at[parity, b]], csem, add=True)

        @pl.when(cnt_here)
        def _():
            for b in range(K1):
                pltpu.make_async_copy(
                    ones, cacc.at[didx.at[parity, b]], csem).wait()

        return 0

    lax.fori_loop(0, n_groups, group, 0)
    for b in range(K1):
        pltpu.make_async_copy(
            rows.at[b], acc.at[didx.at[0, 0]], ssem.at[b]).wait()
    plsc.subcore_barrier()

    @pl.when(c == 0)
    def _():
        pltpu.sync_copy(acc.at[sl], oa.at[sl])
        pltpu.sync_copy(cacc.at[sl], outc0.at[sl])

    @pl.when(c == 1)
    def _():
        pltpu.sync_copy(acc.at[sl], ob.at[sl])
        pltpu.sync_copy(cacc.at[sl], outc1.at[sl])


@functools.partial(
    pl.kernel,
    out_type=[
        jax.ShapeDtypeStruct((ACC_ROWS, N_CLASSES), jnp.bfloat16),
        jax.ShapeDtypeStruct((ACC_ROWS, N_CLASSES), jnp.bfloat16),
    ],
    mesh=_mesh(),
    compiler_params=_SC_PARAMS,
    scratch_types=[
        pltpu.VMEM((N_CHUNKS_2, CHUNK), jnp.int32),
        pltpu.VMEM((N_CHUNKS_2, CHUNK), jnp.int32),
        pltpu.VMEM((K2, CHUNK, N_CLASSES), jnp.bfloat16),
        pltpu.VMEM((CHUNK, N_CLASSES), jnp.bfloat16),
        pltpu.VMEM_SHARED((ACC_ROWS, N_CLASSES), jnp.bfloat16),
        pltpu.SemaphoreType.DMA((K2,)),
        pltpu.SemaphoreType.DMA((K2,)),
    ],
)
def _sc_agg2(p, srcp, dstp, out0, out1, sidx, didx, rows, zb, acc, gsem, ssem):
    c = lax.axis_index("c")
    s = lax.axis_index("s")

    _fill(zb, CHUNK, N_CLASSES, 0.0)

    crow = pl.ds((c * NS + s) * N_CHUNKS_2, N_CHUNKS_2)
    pltpu.sync_copy(srcp.at[crow], sidx)
    pltpu.sync_copy(dstp.at[crow], didx)

    row0 = s * ROWS_PER_TILE
    for k in range(ROWS_PER_TILE // CHUNK):
        pltpu.sync_copy(zb, acc.at[pl.ds(row0 + k * CHUNK, CHUNK)])
    plsc.subcore_barrier()

    def group(g, _):
        for b in range(K2):
            i = g * K2 + b

            @pl.when(g > 0)
            def _():
                pltpu.make_async_copy(
                    rows.at[b], acc.at[didx.at[0]], ssem.at[b]).wait()

            pltpu.async_copy(p.at[sidx.at[i]], rows.at[b], gsem.at[b])

        for b in range(K2):
            i = g * K2 + b
            pltpu.make_async_copy(
                p.at[sidx.at[i]], rows.at[b], gsem.at[b]).wait()
            pltpu.async_copy(
                rows.at[b], acc.at[didx.at[i]], ssem.at[b], add=True)
        return 0

    lax.fori_loop(0, N_CHUNKS_2 // K2, group, 0)
    for b in range(K2):
        pltpu.make_async_copy(
            rows.at[b], acc.at[didx.at[0]], ssem.at[b]).wait()
    plsc.subcore_barrier()

    sl = pl.ds(row0, ROWS_PER_TILE)

    @pl.when(c == 0)
    def _():
        pltpu.sync_copy(acc.at[sl], out0.at[sl])

    @pl.when(c == 1)
    def _():
        pltpu.sync_copy(acc.at[sl], out1.at[sl])


BLK = 1000


def _tc_mid_body(oa, ob, c0, c1, x, w1l, w1r, b1, w2l, w2r, b2,
                 p_out, r2_out):
    c = jnp.maximum(c0[:, 0:1] + c1[:, 0:1], 1.0)
    mean = jnp.concatenate([oa[...].astype(jnp.float32),
                            ob[...].astype(jnp.float32)], axis=1) / c
    h = mean @ w1l[...] + x[...] @ w1r[...] + b1[...]
    h = jnp.maximum(h, 0.0)
    p_out[...] = (h @ w2l[...]).astype(jnp.bfloat16)
    r2_out[...] = h @ w2r[...] + b2[...]


def _tc_mid(oa, ob, cnt0, cnt1, x, w1l, w1r, b1, w2l, w2r, b2):
    full = lambda shape: pl.BlockSpec(shape, lambda i: (0, 0))
    rows = lambda shape: pl.BlockSpec(shape, lambda i: (i, 0))
    return pl.pallas_call(
        _tc_mid_body,
        grid=(N_NODES // BLK,),
        in_specs=[
            rows((BLK, 128)), rows((BLK, 128)),
            rows((BLK, L)), rows((BLK, L)), rows((BLK, D_FEAT)),
            full((D_FEAT, HIDDEN)), full((D_FEAT, HIDDEN)), full((1, HIDDEN)),
            full((HIDDEN, N_CLASSES)), full((HIDDEN, N_CLASSES)), full((1, N_CLASSES)),
        ],
        out_specs=[rows((BLK, N_CLASSES)), rows((BLK, N_CLASSES))],
        out_shape=[
            jax.ShapeDtypeStruct((N_NODES, N_CLASSES), jnp.bfloat16),
            jax.ShapeDtypeStruct((N_NODES, N_CLASSES), jnp.float32),
        ],
    )(oa, ob, cnt0, cnt1, x, w1l, w1r, b1, w2l, w2r, b2)


def _tc_final_body(a0, a1, c0, c1, r2, out):
    c = jnp.maximum(c0[:, 0:1] + c1[:, 0:1], 1.0)
    z = (a0[...].astype(jnp.float32) + a1[...].astype(jnp.float32)) / c + r2[...]
    m = jnp.max(z, axis=1, keepdims=True)
    zs = z - m
    out[...] = zs - jnp.log(jnp.sum(jnp.exp(zs), axis=1, keepdims=True))


def _tc_final(a0, a1, cnt0, cnt1, r2):
    rows = lambda shape: pl.BlockSpec(shape, lambda i: (i, 0))
    return pl.pallas_call(
        _tc_final_body,
        grid=(N_NODES // BLK,),
        in_specs=[rows((BLK, N_CLASSES)), rows((BLK, N_CLASSES)),
                  rows((BLK, L)), rows((BLK, L)), rows((BLK, N_CLASSES))],
        out_specs=rows((BLK, N_CLASSES)),
        out_shape=jax.ShapeDtypeStruct((N_NODES, N_CLASSES), jnp.float32),
    )(a0, a1, cnt0, cnt1, r2)


def kernel(x, edge_index, W1l, W1r, b1, W2l, W2r, b2):
    src = edge_index[0].astype(jnp.int32)
    dst = edge_index[1].astype(jnp.int32)
    n_pad = E_PAD - N_EDGES
    srcp = jnp.concatenate([src, jnp.zeros((n_pad,), jnp.int32)])
    dstp = jnp.concatenate([dst, jnp.full((n_pad,), JUNK_ROW, jnp.int32)])
    srcp = srcp.reshape(E_PAD // CHUNK, CHUNK)
    dstp = dstp.reshape(E_PAD // CHUNK, CHUNK)

    x_bf = x.astype(jnp.bfloat16)
    xa = x_bf[:, :128]
    xb = x_bf[:, 128:]
    oa, ob, cnt0, cnt1 = _sc_agg1(xa, xb, srcp, dstp)

    p, r2 = _tc_mid(oa, ob, cnt0, cnt1, x, W1l, W1r,
                    b1.reshape(1, -1), W2l, W2r, b2.reshape(1, -1))

    a0, a1 = _sc_agg2(p, srcp, dstp)
    return _tc_final(a0, a1, cnt0, cnt1, r2)


# speedup vs baseline: 1.7898x; 1.0045x over previous
import functools

import jax
import jax.numpy as jnp
from jax import lax
from jax.experimental import pallas as pl
from jax.experimental.pallas import tpu as pltpu
from jax.experimental.pallas import tpu_sc as plsc

N_NODES = 10000
D_FEAT = 256
HIDDEN = 256
N_CLASSES = 64
N_EDGES = 160000

NC, NS, L = 2, 16, 16
CHUNK = 128
E_PAD = 163840
ACC_ROWS = 10240
JUNK_ROW = 10008
ROWS_PER_TILE = ACC_ROWS // NS
ACC1_ROWS = 10016
ROWS1_PER_TILE = ACC1_ROWS // NS
CW = 64
K1 = 8
K2 = 8

N_CHUNKS_1 = E_PAD // NS // CHUNK
N_CHUNKS_2 = E_PAD // (NC * NS) // CHUNK


def _fill(ref, n_rows, n_cols, val):
    lanes = 2 * L if ref.dtype == jnp.bfloat16 else L
    groups = n_cols // lanes

    def body(i, _):
        r = i // groups
        g = i % groups
        ref[r, pl.ds(g * lanes, lanes)] = jnp.full((lanes,), val, ref.dtype)
        return 0

    lax.fori_loop(0, n_rows * groups, body, 0)


def _mesh():
    return plsc.VectorSubcoreMesh(core_axis_name="c", subcore_axis_name="s")


_SC_PARAMS = pltpu.CompilerParams(use_tc_tiling_on_sc=False)


@functools.partial(
    pl.kernel,
    out_type=[
        jax.ShapeDtypeStruct((ACC1_ROWS, 128), jnp.bfloat16),
        jax.ShapeDtypeStruct((ACC1_ROWS, 128), jnp.bfloat16),
        jax.ShapeDtypeStruct((ACC1_ROWS, L), jnp.float32),
        jax.ShapeDtypeStruct((ACC1_ROWS, L), jnp.float32),
    ],
    mesh=_mesh(),
    compiler_params=_SC_PARAMS,
    scratch_types=[
        pltpu.VMEM((2, K1, CHUNK), jnp.int32),
        pltpu.VMEM((2, K1, CHUNK), jnp.int32),
        pltpu.VMEM((K1, CHUNK, 128), jnp.bfloat16),
        pltpu.VMEM((CHUNK, L), jnp.float32),
        pltpu.VMEM_SHARED((ACC1_ROWS, 128), jnp.bfloat16),
        pltpu.VMEM_SHARED((ACC1_ROWS, L), jnp.float32),
        pltpu.SemaphoreType.DMA((2,)),
        pltpu.SemaphoreType.DMA((2,)),
        pltpu.SemaphoreType.DMA((K1,)),
        pltpu.SemaphoreType.DMA((K1,)),
        pltpu.SemaphoreType.DMA,
    ],
)
def _sc_agg1(xa, xb, srcp, dstp, oa, ob, outc0, outc1,
             sidx, didx, rows, ones, acc, cacc, isems, idems, gsem, ssem, csem):
    c = lax.axis_index("c")
    s = lax.axis_index("s")

    row0 = s * ROWS1_PER_TILE
    sl = pl.ds(row0, ROWS1_PER_TILE)
    n_chunks = E_PAD // NS // CHUNK
    n_groups = n_chunks // K1
    half = n_groups // 2
    crow0 = s * n_chunks

    _fill(ones, CHUNK, L, 0.0)
    _fill(rows.at[0], CHUNK, 128, 0.0)
    for k in range(4):
        pltpu.sync_copy(rows.at[0], acc.at[pl.ds(row0 + k * CHUNK, CHUNK)])
        pltpu.sync_copy(ones, cacc.at[pl.ds(row0 + k * CHUNK, CHUNK)])
    pltpu.sync_copy(rows.at[0, pl.ds(0, ROWS1_PER_TILE - 4 * CHUNK)],
                    acc.at[pl.ds(row0 + 4 * CHUNK, ROWS1_PER_TILE - 4 * CHUNK)])
    pltpu.sync_copy(ones.at[pl.ds(0, ROWS1_PER_TILE - 4 * CHUNK)],
                    cacc.at[pl.ds(row0 + 4 * CHUNK, ROWS1_PER_TILE - 4 * CHUNK)])
    _fill(ones, CHUNK, L, 1.0)

    pltpu.async_copy(srcp.at[pl.ds(crow0, K1)], sidx.at[0], isems.at[0])
    pltpu.async_copy(dstp.at[pl.ds(crow0, K1)], didx.at[0], idems.at[0])
    plsc.subcore_barrier()

    def group(g, _):
        parity = lax.rem(g, 2)
        nxt = 1 - parity
        cnt_here = ((c == 0) & (g < half)) | ((c == 1) & (g >= half))

        @pl.when(g > 0)
        def _():
            for b in range(K1):
                pltpu.make_async_copy(
                    rows.at[b], acc.at[didx.at[0, 0]], ssem.at[b]).wait()

        @pl.when(g + 1 < n_groups)
        def _():
            r = crow0 + (g + 1) * K1
            pltpu.async_copy(srcp.at[pl.ds(r, K1)], sidx.at[nxt], isems.at[nxt])
            pltpu.async_copy(dstp.at[pl.ds(r, K1)], didx.at[nxt], idems.at[nxt])

        pltpu.make_async_copy(
            srcp.at[pl.ds(0, K1)], sidx.at[parity], isems.at[parity]).wait()
        pltpu.make_async_copy(
            dstp.at[pl.ds(0, K1)], didx.at[parity], idems.at[parity]).wait()

        for b in range(K1):
            @pl.when(c == 0)
            def _():
                pltpu.async_copy(
                    xa.at[sidx.at[parity, b]], rows.at[b], gsem.at[b])

            @pl.when(c == 1)
            def _():
                pltpu.async_copy(
                    xb.at[sidx.at[parity, b]], rows.at[b], gsem.at[b])

        for b in range(K1):
            pltpu.make_async_copy(
                xa.at[sidx.at[parity, b]], rows.at[b], gsem.at[b]).wait()
            pltpu.async_copy(
                rows.at[b], acc.at[didx.at[parity, b]], ssem.at[b], add=True)

            @pl.when(cnt_here)
            def _():
                pltpu.async_copy(
                    ones, cacc.at[didx.at[parity, b]], csem, add=True)

        @pl.when(cnt_here)
        def _():
            for b in range(K1):
                pltpu.make_async_copy(
                    ones, cacc.at[didx.at[parity, b]], csem).wait()

        return 0

    lax.fori_loop(0, n_groups, group, 0)
    for b in range(K1):
        pltpu.make_async_copy(
            rows.at[b], acc.at[didx.at[0, 0]], ssem.at[b]).wait()
    plsc.subcore_barrier()

    @pl.when(c == 0)
    def _():
        pltpu.sync_copy(acc.at[sl], oa.at[sl])
        pltpu.sync_copy(cacc.at[sl], outc0.at[sl])

    @pl.when(c == 1)
    def _():
        pltpu.sync_copy(acc.at[sl], ob.at[sl])
        pltpu.sync_copy(cacc.at[sl], outc1.at[sl])


@functools.partial(
    pl.kernel,
    out_type=[
        jax.ShapeDtypeStruct((ACC_ROWS, N_CLASSES), jnp.bfloat16),
        jax.ShapeDtypeStruct((ACC_ROWS, N_CLASSES), jnp.bfloat16),
    ],
    mesh=_mesh(),
    compiler_params=_SC_PARAMS,
    scratch_types=[
        pltpu.VMEM((N_CHUNKS_2, CHUNK), jnp.int32),
        pltpu.VMEM((N_CHUNKS_2, CHUNK), jnp.int32),
        pltpu.VMEM((K2, CHUNK, N_CLASSES), jnp.bfloat16),
        pltpu.VMEM((CHUNK, N_CLASSES), jnp.bfloat16),
        pltpu.VMEM_SHARED((ACC_ROWS, N_CLASSES), jnp.bfloat16),
        pltpu.SemaphoreType.DMA((K2,)),
        pltpu.SemaphoreType.DMA((K2,)),
    ],
)
def _sc_agg2(p, srcp, dstp, out0, out1, sidx, didx, rows, zb, acc, gsem, ssem):
    c = lax.axis_index("c")
    s = lax.axis_index("s")

    _fill(zb, CHUNK, N_CLASSES, 0.0)

    crow = pl.ds((c * NS + s) * N_CHUNKS_2, N_CHUNKS_2)
    pltpu.sync_copy(srcp.at[crow], sidx)
    pltpu.sync_copy(dstp.at[crow], didx)

    row0 = s * ROWS_PER_TILE
    for k in range(ROWS_PER_TILE // CHUNK):
        pltpu.sync_copy(zb, acc.at[pl.ds(row0 + k * CHUNK, CHUNK)])
    plsc.subcore_barrier()

    def group(g, _):
        for b in range(K2):
            i = g * K2 + b

            @pl.when(g > 0)
            def _():
                pltpu.make_async_copy(
                    rows.at[b], acc.at[didx.at[0]], ssem.at[b]).wait()

            pltpu.async_copy(p.at[sidx.at[i]], rows.at[b], gsem.at[b])

        for b in range(K2):
            i = g * K2 + b
            pltpu.make_async_copy(
                p.at[sidx.at[i]], rows.at[b], gsem.at[b]).wait()
            pltpu.async_copy(
                rows.at[b], acc.at[didx.at[i]], ssem.at[b], add=True)
        return 0

    lax.fori_loop(0, N_CHUNKS_2 // K2, group, 0)
    for b in range(K2):
        pltpu.make_async_copy(
            rows.at[b], acc.at[didx.at[0]], ssem.at[b]).wait()
    plsc.subcore_barrier()

    sl = pl.ds(row0, ROWS_PER_TILE)

    @pl.when(c == 0)
    def _():
        pltpu.sync_copy(acc.at[sl], out0.at[sl])

    @pl.when(c == 1)
    def _():
        pltpu.sync_copy(acc.at[sl], out1.at[sl])


BLK = 1000


def _tc_mid_body(oa, ob, c0, c1, x, w1l, w1r, b1, w2l, w2r, b2,
                 p_out, r2_out):
    c = jnp.maximum(c0[:, 0:1] + c1[:, 0:1], 1.0)
    mean = jnp.concatenate([oa[...].astype(jnp.float32),
                            ob[...].astype(jnp.float32)], axis=1) / c
    h = mean @ w1l[...] + (x[...] @ w1r[...]).astype(jnp.float32) + b1[...]
    h = jnp.maximum(h, 0.0)
    p_out[...] = (h @ w2l[...]).astype(jnp.bfloat16)
    r2_out[...] = h @ w2r[...] + b2[...]


def _tc_mid(oa, ob, cnt0, cnt1, x, w1l, w1r, b1, w2l, w2r, b2):
    full = lambda shape: pl.BlockSpec(shape, lambda i: (0, 0))
    rows = lambda shape: pl.BlockSpec(shape, lambda i: (i, 0))
    return pl.pallas_call(
        _tc_mid_body,
        grid=(N_NODES // BLK,),
        in_specs=[
            rows((BLK, 128)), rows((BLK, 128)),
            rows((BLK, L)), rows((BLK, L)), rows((BLK, D_FEAT)),
            full((D_FEAT, HIDDEN)), full((D_FEAT, HIDDEN)), full((1, HIDDEN)),
            full((HIDDEN, N_CLASSES)), full((HIDDEN, N_CLASSES)), full((1, N_CLASSES)),
        ],
        out_specs=[rows((BLK, N_CLASSES)), rows((BLK, N_CLASSES))],
        out_shape=[
            jax.ShapeDtypeStruct((N_NODES, N_CLASSES), jnp.bfloat16),
            jax.ShapeDtypeStruct((N_NODES, N_CLASSES), jnp.float32),
        ],
    )(oa, ob, cnt0, cnt1, x, w1l, w1r, b1, w2l, w2r, b2)


def _tc_final_body(a0, a1, c0, c1, r2, out):
    c = jnp.maximum(c0[:, 0:1] + c1[:, 0:1], 1.0)
    z = (a0[...].astype(jnp.float32) + a1[...].astype(jnp.float32)) / c + r2[...]
    m = jnp.max(z, axis=1, keepdims=True)
    zs = z - m
    out[...] = zs - jnp.log(jnp.sum(jnp.exp(zs), axis=1, keepdims=True))


def _tc_final(a0, a1, cnt0, cnt1, r2):
    rows = lambda shape: pl.BlockSpec(shape, lambda i: (i, 0))
    return pl.pallas_call(
        _tc_final_body,
        grid=(N_NODES // BLK,),
        in_specs=[rows((BLK, N_CLASSES)), rows((BLK, N_CLASSES)),
                  rows((BLK, L)), rows((BLK, L)), rows((BLK, N_CLASSES))],
        out_specs=rows((BLK, N_CLASSES)),
        out_shape=jax.ShapeDtypeStruct((N_NODES, N_CLASSES), jnp.float32),
    )(a0, a1, cnt0, cnt1, r2)


def kernel(x, edge_index, W1l, W1r, b1, W2l, W2r, b2):
    src = edge_index[0].astype(jnp.int32)
    dst = edge_index[1].astype(jnp.int32)
    n_pad = E_PAD - N_EDGES
    srcp = jnp.concatenate([src, jnp.zeros((n_pad,), jnp.int32)])
    dstp = jnp.concatenate([dst, jnp.full((n_pad,), JUNK_ROW, jnp.int32)])
    srcp = srcp.reshape(E_PAD // CHUNK, CHUNK)
    dstp = dstp.reshape(E_PAD // CHUNK, CHUNK)

    x_bf = x.astype(jnp.bfloat16)
    xa = x_bf[:, :128]
    xb = x_bf[:, 128:]
    oa, ob, cnt0, cnt1 = _sc_agg1(xa, xb, srcp, dstp)

    p, r2 = _tc_mid(oa, ob, cnt0, cnt1, x_bf, W1l, W1r,
                    b1.reshape(1, -1), W2l, W2r, b2.reshape(1, -1))

    a0, a1 = _sc_agg2(p, srcp, dstp)
    return _tc_final(a0, a1, cnt0, cnt1, r2)
